# baseline (speedup 1.0000x reference)
import functools

import jax
import jax.numpy as jnp
from jax import lax
from jax.experimental import pallas as pl
from jax.experimental.pallas import tpu as pltpu
from jax.experimental.pallas import tpu_sc as plsc

_N_USERS = 10000
_CF_LAMBDA = 1e-05
_SLOPE = 0.01

_NC = 2
_NS = 16
_CH = 128


def _scale_chunk(gbuf, vals_v, j, nvreg):
    def body(k, _):
        v16 = vals_v[pl.ds(j * _CH + k * 16, 16)]
        for i in range(16):
            r = j * _CH + k * 16 + i
            v = v16[i]
            for q in range(nvreg):
                g = gbuf[r, pl.ds(q * 16, 16)]
                gbuf[r, pl.ds(q * 16, 16)] = g * v
        return 0

    lax.fori_loop(0, _CH // 16, body, 0)


def _zero_gbuf(gbuf, be, nvreg):
    z = jnp.zeros((16,), jnp.float32)

    def zb(i, _):
        for q in range(nvreg):
            gbuf[i, pl.ds(q * 16, 16)] = z
        return 0

    lax.fori_loop(0, be, zb, 0)


def _fill_zero(gbuf, acc, base, total, be):
    done = 0
    while done < total:
        step = min(be, total - done)
        pltpu.sync_copy(gbuf.at[pl.ds(0, step), :], acc.at[pl.ds(base + done, step), :])
        done += step


def _tile_row_split(N):
    ra = ((N // _NS + 7) // 8) * 8
    rlast = N - (_NS - 1) * ra
    assert rlast > 0 and rlast % 8 == 0
    return ra, rlast


def _make_seg0(N, E, Dh):
    nvreg = Dh // 16
    EP = E // _NS
    CPB = 5
    BE = CPB * _CH
    NB = EP // BE
    assert EP % BE == 0
    RA, RLAST = _tile_row_split(N)
    CROWS = EP // _CH

    mesh = plsc.VectorSubcoreMesh(core_axis_name="c", subcore_axis_name="s")

    @functools.partial(
        pl.kernel,
        mesh=mesh,
        out_type=(
            jax.ShapeDtypeStruct((N, Dh), jnp.float32),
            jax.ShapeDtypeStruct((N, Dh), jnp.float32),
        ),
        scratch_types=[
            pltpu.VMEM((BE,), jnp.int32),
            pltpu.VMEM((BE,), jnp.int32),
            pltpu.VMEM((BE,), jnp.float32),
            pltpu.VMEM((BE, Dh), jnp.float32),
            pltpu.MemorySpace.VMEM_SHARED((N, Dh), jnp.float32),
            pltpu.SemaphoreType.DMA,
            pltpu.SemaphoreType.DMA,
        ],
        compiler_params=pltpu.CompilerParams(use_tc_tiling_on_sc=False),
    )
    def k(lo_hbm, hi_hbm, cols_hbm, rows_hbm, vals_hbm, out_lo, out_hi,
          cols_v, rows_v, vals_v, gbuf, acc, gsem, ssem):
        c = lax.axis_index("c")
        s = lax.axis_index("s")

        def run(tab, out):
            _zero_gbuf(gbuf, BE, nvreg)
            rbase = pl.multiple_of(s * RA, 8)

            @pl.when(s < _NS - 1)
            def _():
                _fill_zero(gbuf, acc, rbase, RA, BE)

            @pl.when(s == _NS - 1)
            def _():
                _fill_zero(gbuf, acc, rbase, RLAST, BE)

            plsc.subcore_barrier()

            def blk(b, _):
                e0 = (s * CROWS + b * CPB) * _CH
                pltpu.sync_copy(cols_hbm.at[pl.ds(e0, BE)], cols_v)
                pltpu.sync_copy(rows_hbm.at[pl.ds(e0, BE)], rows_v)
                pltpu.sync_copy(vals_hbm.at[pl.ds(e0, BE)], vals_v)
                gds = [
                    pltpu.async_copy(
                        tab.at[cols_v.at[pl.ds(j * _CH, _CH)]],
                        gbuf.at[pl.ds(j * _CH, _CH), :],
                        gsem,
                    )
                    for j in range(CPB)
                ]
                sds = []
                for j in range(CPB):
                    gds[j].wait()
                    _scale_chunk(gbuf, vals_v, j, nvreg)
                    sds.append(
                        pltpu.async_copy(
                            gbuf.at[pl.ds(j * _CH, _CH), :],
                            acc.at[rows_v.at[pl.ds(j * _CH, _CH)]],
                            ssem,
                            add=True,
                        )
                    )
                for d in sds:
                    d.wait()
                return 0

            lax.fori_loop(0, NB, blk, 0)
            plsc.subcore_barrier()

            @pl.when(s < _NS - 1)
            def _():
                pltpu.sync_copy(acc.at[pl.ds(rbase, RA), :], out.at[pl.ds(rbase, RA), :])

            @pl.when(s == _NS - 1)
            def _():
                pltpu.sync_copy(acc.at[pl.ds(rbase, RLAST), :], out.at[pl.ds(rbase, RLAST), :])

        @pl.when(c == 0)
        def _():
            run(lo_hbm, out_lo)

        @pl.when(c == 1)
        def _():
            run(hi_hbm, out_hi)

    return k


def _make_seg1(N, E, Din):
    nvreg = Din // 16
    NW = _NC * _NS
    EP = E // NW
    CPB = 5
    BE = CPB * _CH
    NB = EP // BE
    assert EP % BE == 0
    RA, RLAST = _tile_row_split(N)
    CROWS = EP // _CH

    mesh = plsc.VectorSubcoreMesh(core_axis_name="c", subcore_axis_name="s")

    @functools.partial(
        pl.kernel,
        mesh=mesh,
        out_type=(
            jax.ShapeDtypeStruct((N, Din), jnp.float32),
            jax.ShapeDtypeStruct((N, Din), jnp.float32),
        ),
        scratch_types=[
            pltpu.VMEM((BE,), jnp.int32),
            pltpu.VMEM((BE,), jnp.int32),
            pltpu.VMEM((BE,), jnp.float32),
            pltpu.VMEM((BE, Din), jnp.float32),
            pltpu.MemorySpace.VMEM_SHARED((N, Din), jnp.float32),
            pltpu.SemaphoreType.DMA,
            pltpu.SemaphoreType.DMA,
        ],
        compiler_params=pltpu.CompilerParams(use_tc_tiling_on_sc=False),
    )
    def k(tab_hbm, cols_hbm, rows_hbm, vals_hbm, out_a, out_b,
          cols_v, rows_v, vals_v, gbuf, acc, gsem, ssem):
        c = lax.axis_index("c")
        s = lax.axis_index("s")
        wid = s * _NC + c

        _zero_gbuf(gbuf, BE, nvreg)
        rbase = pl.multiple_of(s * RA, 8)

        @pl.when(s < _NS - 1)
        def _():
            _fill_zero(gbuf, acc, rbase, RA, BE)

        @pl.when(s == _NS - 1)
        def _():
            _fill_zero(gbuf, acc, rbase, RLAST, BE)

        plsc.subcore_barrier()

        def blk(b, _):
            e0 = (wid * CROWS + b * CPB) * _CH
            pltpu.sync_copy(cols_hbm.at[pl.ds(e0, BE)], cols_v)
            pltpu.sync_copy(rows_hbm.at[pl.ds(e0, BE)], rows_v)
            pltpu.sync_copy(vals_hbm.at[pl.ds(e0, BE)], vals_v)
            gds = [
                pltpu.async_copy(
                    tab_hbm.at[cols_v.at[pl.ds(j * _CH, _CH)]],
                    gbuf.at[pl.ds(j * _CH, _CH), :],
                    gsem,
                )
                for j in range(CPB)
            ]
            sds = []
            for j in range(CPB):
                gds[j].wait()
                _scale_chunk(gbuf, vals_v, j, nvreg)
                sds.append(
                    pltpu.async_copy(
                        gbuf.at[pl.ds(j * _CH, _CH), :],
                        acc.at[rows_v.at[pl.ds(j * _CH, _CH)]],
                        ssem,
                        add=True,
                    )
                )
            for d in sds:
                d.wait()
            return 0

        lax.fori_loop(0, NB, blk, 0)
        plsc.subcore_barrier()

        def wb(out):
            @pl.when(s < _NS - 1)
            def _():
                pltpu.sync_copy(acc.at[pl.ds(rbase, RA), :], out.at[pl.ds(rbase, RA), :])

            @pl.when(s == _NS - 1)
            def _():
                pltpu.sync_copy(acc.at[pl.ds(rbase, RLAST), :], out.at[pl.ds(rbase, RLAST), :])

        @pl.when(c == 0)
        def _():
            wb(out_a)

        @pl.when(c == 1)
        def _():
            wb(out_b)

    return k


def _make_gather(N, B):
    NW = _NC * _NS
    per_w = B // NW
    nch = per_w // 128
    assert per_w % 128 == 0

    mesh = plsc.VectorSubcoreMesh(core_axis_name="c", subcore_axis_name="s")

    @functools.partial(
        pl.kernel,
        mesh=mesh,
        out_type=(
            jax.ShapeDtypeStruct((B, 64), jnp.float32),
            jax.ShapeDtypeStruct((B, 32), jnp.float32),
            jax.ShapeDtypeStruct((B, 16), jnp.float32),
        ),
        scratch_types=[
            pltpu.VMEM((nch, 128), jnp.int32),
            pltpu.VMEM((per_w, 64), jnp.float32),
            pltpu.VMEM((per_w, 32), jnp.float32),
            pltpu.VMEM((per_w, 16), jnp.float32),
            pltpu.SemaphoreType.DMA,
        ],
        compiler_params=pltpu.CompilerParams(use_tc_tiling_on_sc=False),
    )
    def k(t64, t32, t16, idx_hbm, o64, o32, o16, idx_v, b64, b32, b16, sem):
        c = lax.axis_index("c")
        s = lax.axis_index("s")
        wid = s * _NC + c
        pltpu.sync_copy(idx_hbm.at[wid], idx_v)
        ds_ = []
        for j in range(nch):
            ds_.append(pltpu.async_copy(t64.at[idx_v.at[j]], b64.at[pl.ds(j * 128, 128), :], sem))
            ds_.append(pltpu.async_copy(t32.at[idx_v.at[j]], b32.at[pl.ds(j * 128, 128), :], sem))
            ds_.append(pltpu.async_copy(t16.at[idx_v.at[j]], b16.at[pl.ds(j * 128, 128), :], sem))
        for d in ds_:
            d.wait()
        base = pl.multiple_of(wid * per_w, 8)
        pltpu.sync_copy(b64, o64.at[pl.ds(base, per_w), :])
        pltpu.sync_copy(b32, o32.at[pl.ds(base, per_w), :])
        pltpu.sync_copy(b16, o16.at[pl.ds(base, per_w), :])

    return k


def _leaky(x):
    return jnp.where(x > 0, x, x * _SLOPE)


def _dense0_body(ego, nlo, nhi, wg, bg, wb, bb, ego1_out, emb1_out):
    ego = ego[...]
    neigh = jnp.concatenate([nlo[...], nhi[...]], axis=1)
    x1 = ego + neigh
    x2 = ego * neigh
    a = _leaky(jnp.dot(x1, wg[...], preferred_element_type=jnp.float32) + bg[...])
    w = _leaky(jnp.dot(x2, wb[...], preferred_element_type=jnp.float32) + bb[...])
    e1 = a + w
    ego1_out[...] = e1
    nrm = jnp.sqrt(jnp.sum(e1 * e1, axis=1, keepdims=True))
    emb1_out[...] = e1 / jnp.maximum(nrm, 1e-12)


def _dense1_body(ego1, na, nb_, wg, bg, wb, bb, emb2_out):
    ego1 = ego1[...]
    neigh = na[...] + nb_[...]
    x1 = ego1 + neigh
    x2 = ego1 * neigh
    a = _leaky(jnp.dot(x1, wg[...], preferred_element_type=jnp.float32) + bg[...])
    w = _leaky(jnp.dot(x2, wb[...], preferred_element_type=jnp.float32) + bb[...])
    e2 = a + w
    nrm = jnp.sqrt(jnp.sum(e2 * e2, axis=1, keepdims=True))
    emb2_out[...] = e2 / jnp.maximum(nrm, 1e-12)


def _loss_body(g64, g32, g16, out):
    B = g64.shape[0] // 3
    u64, p64, n64 = g64[:B], g64[B:2 * B], g64[2 * B:]
    u32, p32, n32 = g32[:B], g32[B:2 * B], g32[2 * B:]
    u16, p16, n16 = g16[:B], g16[B:2 * B], g16[2 * B:]
    pos = (jnp.sum(u64 * p64, axis=1, keepdims=True)
           + jnp.sum(u32 * p32, axis=1, keepdims=True)
           + jnp.sum(u16 * p16, axis=1, keepdims=True))
    neg = (jnp.sum(u64 * n64, axis=1, keepdims=True)
           + jnp.sum(u32 * n32, axis=1, keepdims=True)
           + jnp.sum(u16 * n16, axis=1, keepdims=True))
    d = pos - neg
    sp = jnp.maximum(-d, 0.0) + jnp.log1p(jnp.exp(-jnp.abs(d)))
    cf = jnp.sum(sp) / B
    sq_u = (jnp.sum(u64 * u64, axis=1, keepdims=True)
            + jnp.sum(u32 * u32, axis=1, keepdims=True)
            + jnp.sum(u16 * u16, axis=1, keepdims=True))
    sq_p = (jnp.sum(p64 * p64, axis=1, keepdims=True)
            + jnp.sum(p32 * p32, axis=1, keepdims=True)
            + jnp.sum(p16 * p16, axis=1, keepdims=True))
    sq_n = (jnp.sum(n64 * n64, axis=1, keepdims=True)
            + jnp.sum(n32 * n32, axis=1, keepdims=True)
            + jnp.sum(n16 * n16, axis=1, keepdims=True))
    l2 = (jnp.sum(sq_u) + jnp.sum(sq_p) + jnp.sum(sq_n)) / (2.0 * B)
    out[...] = jnp.reshape(cf + _CF_LAMBDA * l2, (1, 1))


def _dense0_call(N):
    R = 2000
    grid = (N // R,)
    full = lambda i: (0, 0)
    blk = lambda i: (i, 0)

    return pl.pallas_call(
        _dense0_body,
        grid=grid,
        in_specs=[
            pl.BlockSpec((R, 64), blk),
            pl.BlockSpec((R, 32), blk),
            pl.BlockSpec((R, 32), blk),
            pl.BlockSpec((64, 32), full),
            pl.BlockSpec((1, 32), full),
            pl.BlockSpec((64, 32), full),
            pl.BlockSpec((1, 32), full),
        ],
        out_specs=[
            pl.BlockSpec((R, 32), blk),
            pl.BlockSpec((R, 32), blk),
        ],
        out_shape=[
            jax.ShapeDtypeStruct((N, 32), jnp.float32),
            jax.ShapeDtypeStruct((N, 32), jnp.float32),
        ],
    )


def _dense1_call(N):
    R = 2000
    grid = (N // R,)
    full = lambda i: (0, 0)
    blk = lambda i: (i, 0)

    return pl.pallas_call(
        _dense1_body,
        grid=grid,
        in_specs=[
            pl.BlockSpec((R, 32), blk),
            pl.BlockSpec((R, 32), blk),
            pl.BlockSpec((R, 32), blk),
            pl.BlockSpec((32, 16), full),
            pl.BlockSpec((1, 16), full),
            pl.BlockSpec((32, 16), full),
            pl.BlockSpec((1, 16), full),
        ],
        out_specs=pl.BlockSpec((R, 16), blk),
        out_shape=jax.ShapeDtypeStruct((N, 16), jnp.float32),
    )


def _loss_call():
    def body(g64, g32, g16, out):
        _loss_body(g64[...], g32[...], g16[...], out)

    return pl.pallas_call(
        body,
        out_shape=jax.ShapeDtypeStruct((1, 1), jnp.float32),
    )


def kernel(all_emb, W_gc0, b_gc0, W_bi0, b_bi0, W_gc1, b_gc1, W_bi1, b_bi1,
           edge_vals, user_ids, item_pos_ids, item_neg_ids, edge_index):
    N, D = all_emb.shape
    E = edge_vals.shape[0]
    Dh = D // 2

    align = _NC * _NS * _CH * 8
    Ep = ((E + align - 1) // align) * align
    pad = Ep - E
    rows = edge_index[0].astype(jnp.int32)
    cols = edge_index[1].astype(jnp.int32)
    if pad:
        spread = (jnp.arange(pad, dtype=jnp.int32) * 97) % jnp.int32(N)
        rows = jnp.concatenate([rows, spread])
        cols = jnp.concatenate([cols, spread])
        vals = jnp.concatenate([edge_vals, jnp.zeros((pad,), jnp.float32)])
    else:
        vals = edge_vals
    ego_lo = all_emb[:, :Dh]
    ego_hi = all_emb[:, Dh:]

    n_lo, n_hi = _make_seg0(N, Ep, Dh)(ego_lo, ego_hi, cols, rows, vals)
    ego1, emb1 = _dense0_call(N)(
        all_emb, n_lo, n_hi,
        W_gc0, b_gc0.reshape(1, -1), W_bi0, b_bi0.reshape(1, -1))

    n1a, n1b = _make_seg1(N, Ep, 32)(ego1, cols, rows, vals)
    emb2 = _dense1_call(N)(
        ego1, n1a, n1b,
        W_gc1, b_gc1.reshape(1, -1), W_bi1, b_bi1.reshape(1, -1))

    B = user_ids.shape[0]
    idx = jnp.concatenate([
        user_ids.astype(jnp.int32),
        item_pos_ids.astype(jnp.int32) + _N_USERS,
        item_neg_ids.astype(jnp.int32) + _N_USERS,
    ]).reshape(_NC * _NS, (3 * B) // (_NC * _NS * 128), 128)

    g64, g32, g16 = _make_gather(N, 3 * B)(all_emb, emb1, emb2, idx)
    loss = _loss_call()(g64, g32, g16)
    return loss.reshape(())


# speedup vs baseline: 1.1049x; 1.1049x over previous
import functools

import jax
import jax.numpy as jnp
from jax import lax
from jax.experimental import pallas as pl
from jax.experimental.pallas import tpu as pltpu
from jax.experimental.pallas import tpu_sc as plsc

_N_USERS = 10000
_CF_LAMBDA = 1e-05
_SLOPE = 0.01

_NC = 2
_NS = 16
_CH = 128


def _scale_chunk(gbuf, vals_v, j, nvreg):
    def body(k, _):
        v16 = vals_v[pl.ds(j * _CH + k * 16, 16)]
        for i in range(16):
            r = j * _CH + k * 16 + i
            v = v16[i]
            for q in range(nvreg):
                g = gbuf[r, pl.ds(q * 16, 16)]
                gbuf[r, pl.ds(q * 16, 16)] = g * v
        return 0

    lax.fori_loop(0, _CH // 16, body, 0)


def _zero_gbuf(gbuf, be, nvreg):
    z = jnp.zeros((16,), jnp.float32)

    def zb(i, _):
        for q in range(nvreg):
            gbuf[i, pl.ds(q * 16, 16)] = z
        return 0

    lax.fori_loop(0, be, zb, 0)


def _fill_zero(gbuf, acc, base, total, be):
    done = 0
    while done < total:
        step = min(be, total - done)
        pltpu.sync_copy(gbuf.at[pl.ds(0, step), :], acc.at[pl.ds(base + done, step), :])
        done += step


def _tile_row_split(N):
    ra = ((N // _NS + 7) // 8) * 8
    rlast = N - (_NS - 1) * ra
    assert rlast > 0 and rlast % 8 == 0
    return ra, rlast


def _make_seg0(N, E, Dh):
    nvreg = Dh // 16
    EP = E // _NS
    CPB = 5
    BE = CPB * _CH
    NB = EP // BE
    assert EP % BE == 0
    RA, RLAST = _tile_row_split(N)
    CROWS = EP // _CH

    mesh = plsc.VectorSubcoreMesh(core_axis_name="c", subcore_axis_name="s")

    @functools.partial(
        pl.kernel,
        mesh=mesh,
        out_type=(
            jax.ShapeDtypeStruct((N, Dh), jnp.float32),
            jax.ShapeDtypeStruct((N, Dh), jnp.float32),
        ),
        scratch_types=[
            pltpu.VMEM((BE,), jnp.int32),
            pltpu.VMEM((BE,), jnp.int32),
            pltpu.VMEM((BE,), jnp.float32),
            pltpu.VMEM((BE, Dh), jnp.float32),
            pltpu.MemorySpace.VMEM_SHARED((N, Dh), jnp.float32),
            pltpu.SemaphoreType.DMA,
            pltpu.SemaphoreType.DMA,
        ],
        compiler_params=pltpu.CompilerParams(use_tc_tiling_on_sc=False),
    )
    def k(lo_hbm, hi_hbm, cols_hbm, rows_hbm, vals_hbm, out_lo, out_hi,
          cols_v, rows_v, vals_v, gbuf, acc, gsem, ssem):
        c = lax.axis_index("c")
        s = lax.axis_index("s")

        def run(tab, out):
            _zero_gbuf(gbuf, BE, nvreg)
            rbase = pl.multiple_of(s * RA, 8)

            @pl.when(s < _NS - 1)
            def _():
                _fill_zero(gbuf, acc, rbase, RA, BE)

            @pl.when(s == _NS - 1)
            def _():
                _fill_zero(gbuf, acc, rbase, RLAST, BE)

            plsc.subcore_barrier()

            def blk(b, _):
                e0 = (s * CROWS + b * CPB) * _CH
                pltpu.sync_copy(cols_hbm.at[pl.ds(e0, BE)], cols_v)
                pltpu.sync_copy(rows_hbm.at[pl.ds(e0, BE)], rows_v)
                pltpu.sync_copy(vals_hbm.at[pl.ds(e0, BE)], vals_v)
                gds = [
                    pltpu.async_copy(
                        tab.at[cols_v.at[pl.ds(j * _CH, _CH)]],
                        gbuf.at[pl.ds(j * _CH, _CH), :],
                        gsem,
                    )
                    for j in range(CPB)
                ]
                sds = []
                for j in range(CPB):
                    gds[j].wait()
                    _scale_chunk(gbuf, vals_v, j, nvreg)
                    sds.append(
                        pltpu.async_copy(
                            gbuf.at[pl.ds(j * _CH, _CH), :],
                            acc.at[rows_v.at[pl.ds(j * _CH, _CH)]],
                            ssem,
                            add=True,
                        )
                    )
                for d in sds:
                    d.wait()
                return 0

            lax.fori_loop(0, NB, blk, 0)
            plsc.subcore_barrier()

            @pl.when(s < _NS - 1)
            def _():
                pltpu.sync_copy(acc.at[pl.ds(rbase, RA), :], out.at[pl.ds(rbase, RA), :])

            @pl.when(s == _NS - 1)
            def _():
                pltpu.sync_copy(acc.at[pl.ds(rbase, RLAST), :], out.at[pl.ds(rbase, RLAST), :])

        @pl.when(c == 0)
        def _():
            run(lo_hbm, out_lo)

        @pl.when(c == 1)
        def _():
            run(hi_hbm, out_hi)

    return k


def _make_seg1(N, E, Din):
    nvreg = Din // 16
    NW = _NC * _NS
    EP = E // NW
    CPB = 5
    BE = CPB * _CH
    NB = EP // BE
    assert EP % BE == 0
    RA, RLAST = _tile_row_split(N)
    CROWS = EP // _CH

    mesh = plsc.VectorSubcoreMesh(core_axis_name="c", subcore_axis_name="s")

    @functools.partial(
        pl.kernel,
        mesh=mesh,
        out_type=(
            jax.ShapeDtypeStruct((N, Din), jnp.float32),
            jax.ShapeDtypeStruct((N, Din), jnp.float32),
        ),
        scratch_types=[
            pltpu.VMEM((BE,), jnp.int32),
            pltpu.VMEM((BE,), jnp.int32),
            pltpu.VMEM((BE,), jnp.float32),
            pltpu.VMEM((BE, Din), jnp.float32),
            pltpu.MemorySpace.VMEM_SHARED((N, Din), jnp.float32),
            pltpu.SemaphoreType.DMA,
            pltpu.SemaphoreType.DMA,
        ],
        compiler_params=pltpu.CompilerParams(use_tc_tiling_on_sc=False),
    )
    def k(tab_hbm, cols_hbm, rows_hbm, vals_hbm, out_a, out_b,
          cols_v, rows_v, vals_v, gbuf, acc, gsem, ssem):
        c = lax.axis_index("c")
        s = lax.axis_index("s")
        wid = s * _NC + c

        _zero_gbuf(gbuf, BE, nvreg)
        rbase = pl.multiple_of(s * RA, 8)

        @pl.when(s < _NS - 1)
        def _():
            _fill_zero(gbuf, acc, rbase, RA, BE)

        @pl.when(s == _NS - 1)
        def _():
            _fill_zero(gbuf, acc, rbase, RLAST, BE)

        plsc.subcore_barrier()

        def blk(b, _):
            e0 = (wid * CROWS + b * CPB) * _CH
            pltpu.sync_copy(cols_hbm.at[pl.ds(e0, BE)], cols_v)
            pltpu.sync_copy(rows_hbm.at[pl.ds(e0, BE)], rows_v)
            pltpu.sync_copy(vals_hbm.at[pl.ds(e0, BE)], vals_v)
            gds = [
                pltpu.async_copy(
                    tab_hbm.at[cols_v.at[pl.ds(j * _CH, _CH)]],
                    gbuf.at[pl.ds(j * _CH, _CH), :],
                    gsem,
                )
                for j in range(CPB)
            ]
            sds = []
            for j in range(CPB):
                gds[j].wait()
                _scale_chunk(gbuf, vals_v, j, nvreg)
                sds.append(
                    pltpu.async_copy(
                        gbuf.at[pl.ds(j * _CH, _CH), :],
                        acc.at[rows_v.at[pl.ds(j * _CH, _CH)]],
                        ssem,
                        add=True,
                    )
                )
            for d in sds:
                d.wait()
            return 0

        lax.fori_loop(0, NB, blk, 0)
        plsc.subcore_barrier()

        def wb(out):
            @pl.when(s < _NS - 1)
            def _():
                pltpu.sync_copy(acc.at[pl.ds(rbase, RA), :], out.at[pl.ds(rbase, RA), :])

            @pl.when(s == _NS - 1)
            def _():
                pltpu.sync_copy(acc.at[pl.ds(rbase, RLAST), :], out.at[pl.ds(rbase, RLAST), :])

        @pl.when(c == 0)
        def _():
            wb(out_a)

        @pl.when(c == 1)
        def _():
            wb(out_b)

    return k


def _make_gather(N, B):
    NW = _NC * _NS
    per_w = B // NW
    nch = per_w // 128
    assert per_w % 128 == 0

    mesh = plsc.VectorSubcoreMesh(core_axis_name="c", subcore_axis_name="s")

    @functools.partial(
        pl.kernel,
        mesh=mesh,
        out_type=jax.ShapeDtypeStruct((B, 160), jnp.float32),
        scratch_types=[
            pltpu.VMEM((nch, 128), jnp.int32),
            pltpu.VMEM((per_w, 64), jnp.float32),
            pltpu.VMEM((per_w, 32), jnp.float32),
            pltpu.VMEM((per_w, 32), jnp.float32),
            pltpu.VMEM((per_w, 32), jnp.float32),
            pltpu.SemaphoreType.DMA,
        ],
        compiler_params=pltpu.CompilerParams(use_tc_tiling_on_sc=False),
    )
    def k(t64, te1, tna, tnb, idx_hbm, out, idx_v, b64, be1, bna, bnb, sem):
        c = lax.axis_index("c")
        s = lax.axis_index("s")
        wid = s * _NC + c
        pltpu.sync_copy(idx_hbm.at[wid], idx_v)
        ds_ = []
        for j in range(nch):
            ds_.append(pltpu.async_copy(t64.at[idx_v.at[j]], b64.at[pl.ds(j * 128, 128), :], sem))
            ds_.append(pltpu.async_copy(te1.at[idx_v.at[j]], be1.at[pl.ds(j * 128, 128), :], sem))
            ds_.append(pltpu.async_copy(tna.at[idx_v.at[j]], bna.at[pl.ds(j * 128, 128), :], sem))
            ds_.append(pltpu.async_copy(tnb.at[idx_v.at[j]], bnb.at[pl.ds(j * 128, 128), :], sem))
        for d in ds_:
            d.wait()
        base = pl.multiple_of(wid * per_w, 8)
        pltpu.sync_copy(b64, out.at[pl.ds(base, per_w), pl.ds(0, 64)])
        pltpu.sync_copy(be1, out.at[pl.ds(base, per_w), pl.ds(64, 32)])
        pltpu.sync_copy(bna, out.at[pl.ds(base, per_w), pl.ds(96, 32)])
        pltpu.sync_copy(bnb, out.at[pl.ds(base, per_w), pl.ds(128, 32)])

    return k


def _leaky(x):
    return jnp.where(x > 0, x, x * _SLOPE)


def _dense0_body(ego, nlo, nhi, wg, bg, wb, bb, ego1_out):
    ego = ego[...]
    neigh = jnp.concatenate([nlo[...], nhi[...]], axis=1)
    x1 = ego + neigh
    x2 = ego * neigh
    a = _leaky(jnp.dot(x1, wg[...], preferred_element_type=jnp.float32) + bg[...])
    w = _leaky(jnp.dot(x2, wb[...], preferred_element_type=jnp.float32) + bb[...])
    ego1_out[...] = a + w


def _norm_rows(x):
    nrm = jnp.sqrt(jnp.sum(x * x, axis=1, keepdims=True))
    return x / jnp.maximum(nrm, 1e-12)


def _dense1_loss_body(packed, wg, bg, wb, bb, out):
    x = packed[...]
    g64 = x[:, :64]
    ge1 = x[:, 64:96]
    n1 = x[:, 96:128] + x[:, 128:160]
    g32 = _norm_rows(ge1)
    x1 = ge1 + n1
    x2 = ge1 * n1
    a = _leaky(jnp.dot(x1, wg[...], preferred_element_type=jnp.float32) + bg[...])
    w = _leaky(jnp.dot(x2, wb[...], preferred_element_type=jnp.float32) + bb[...])
    g16 = _norm_rows(a + w)
    B = g64.shape[0] // 3
    u64, p64, n64 = g64[:B], g64[B:2 * B], g64[2 * B:]
    u32, p32, n32 = g32[:B], g32[B:2 * B], g32[2 * B:]
    u16, p16, n16 = g16[:B], g16[B:2 * B], g16[2 * B:]
    pos = (jnp.sum(u64 * p64, axis=1, keepdims=True)
           + jnp.sum(u32 * p32, axis=1, keepdims=True)
           + jnp.sum(u16 * p16, axis=1, keepdims=True))
    neg = (jnp.sum(u64 * n64, axis=1, keepdims=True)
           + jnp.sum(u32 * n32, axis=1, keepdims=True)
           + jnp.sum(u16 * n16, axis=1, keepdims=True))
    d = pos - neg
    sp = jnp.maximum(-d, 0.0) + jnp.log1p(jnp.exp(-jnp.abs(d)))
    cf = jnp.sum(sp) / B
    sq_u = (jnp.sum(u64 * u64, axis=1, keepdims=True)
            + jnp.sum(u32 * u32, axis=1, keepdims=True)
            + jnp.sum(u16 * u16, axis=1, keepdims=True))
    sq_p = (jnp.sum(p64 * p64, axis=1, keepdims=True)
            + jnp.sum(p32 * p32, axis=1, keepdims=True)
            + jnp.sum(p16 * p16, axis=1, keepdims=True))
    sq_n = (jnp.sum(n64 * n64, axis=1, keepdims=True)
            + jnp.sum(n32 * n32, axis=1, keepdims=True)
            + jnp.sum(n16 * n16, axis=1, keepdims=True))
    l2 = (jnp.sum(sq_u) + jnp.sum(sq_p) + jnp.sum(sq_n)) / (2.0 * B)
    out[...] = jnp.reshape(cf + _CF_LAMBDA * l2, (1, 1))


def _dense0_call(N):
    R = 2000
    grid = (N // R,)
    full = lambda i: (0, 0)
    blk = lambda i: (i, 0)

    return pl.pallas_call(
        _dense0_body,
        grid=grid,
        in_specs=[
            pl.BlockSpec((R, 64), blk),
            pl.BlockSpec((R, 32), blk),
            pl.BlockSpec((R, 32), blk),
            pl.BlockSpec((64, 32), full),
            pl.BlockSpec((1, 32), full),
            pl.BlockSpec((64, 32), full),
            pl.BlockSpec((1, 32), full),
        ],
        out_specs=pl.BlockSpec((R, 32), blk),
        out_shape=jax.ShapeDtypeStruct((N, 32), jnp.float32),
    )


def _dense1_loss_call():
    return pl.pallas_call(
        _dense1_loss_body,
        out_shape=jax.ShapeDtypeStruct((1, 1), jnp.float32),
    )


def kernel(all_emb, W_gc0, b_gc0, W_bi0, b_bi0, W_gc1, b_gc1, W_bi1, b_bi1,
           edge_vals, user_ids, item_pos_ids, item_neg_ids, edge_index):
    N, D = all_emb.shape
    E = edge_vals.shape[0]
    Dh = D // 2

    align = _NC * _NS * _CH * 8
    Ep = ((E + align - 1) // align) * align
    pad = Ep - E
    rows = edge_index[0].astype(jnp.int32)
    cols = edge_index[1].astype(jnp.int32)
    if pad:
        spread = (jnp.arange(pad, dtype=jnp.int32) * 97) % jnp.int32(N)
        rows = jnp.concatenate([rows, spread])
        cols = jnp.concatenate([cols, spread])
        vals = jnp.concatenate([edge_vals, jnp.zeros((pad,), jnp.float32)])
    else:
        vals = edge_vals
    ego_lo = all_emb[:, :Dh]
    ego_hi = all_emb[:, Dh:]

    n_lo, n_hi = _make_seg0(N, Ep, Dh)(ego_lo, ego_hi, cols, rows, vals)
    ego1 = _dense0_call(N)(
        all_emb, n_lo, n_hi,
        W_gc0, b_gc0.reshape(1, -1), W_bi0, b_bi0.reshape(1, -1))

    n1a, n1b = _make_seg1(N, Ep, 32)(ego1, cols, rows, vals)

    B = user_ids.shape[0]
    idx = jnp.concatenate([
        user_ids.astype(jnp.int32),
        item_pos_ids.astype(jnp.int32) + _N_USERS,
        item_neg_ids.astype(jnp.int32) + _N_USERS,
    ]).reshape(_NC * _NS, (3 * B) // (_NC * _NS * 128), 128)

    packed = _make_gather(N, 3 * B)(all_emb, ego1, n1a, n1b, idx)
    loss = _dense1_loss_call()(
        packed, W_gc1, b_gc1.reshape(1, -1), W_bi1, b_bi1.reshape(1, -1))
    return loss.reshape(())


# speedup vs baseline: 1.3899x; 1.2579x over previous
import functools

import jax
import jax.numpy as jnp
from jax import lax
from jax.experimental import pallas as pl
from jax.experimental.pallas import tpu as pltpu
from jax.experimental.pallas import tpu_sc as plsc

_N_USERS = 10000
_CF_LAMBDA = 1e-05
_SLOPE = 0.01

_NC = 2
_NS = 16
_CH = 128


def _scale_chunk(gbuf, vals_v, vbase, j, nvreg):
    def body(k, _):
        v16 = vals_v[pl.ds(vbase + j * _CH + k * 16, 16)]
        for i in range(16):
            r = j * _CH + k * 16 + i
            v = v16[i]
            for q in range(nvreg):
                g = gbuf[r, pl.ds(q * 16, 16)]
                gbuf[r, pl.ds(q * 16, 16)] = g * v
        return 0

    lax.fori_loop(0, _CH // 16, body, 0)


def _zero_gbuf(gbuf, be, nvreg):
    z = jnp.zeros((16,), jnp.float32)

    def zb(i, _):
        for q in range(nvreg):
            gbuf[i, pl.ds(q * 16, 16)] = z
        return 0

    lax.fori_loop(0, be, zb, 0)


def _fill_zero(gbuf, acc, base, total, be):
    done = 0
    while done < total:
        step = min(be, total - done)
        pltpu.sync_copy(gbuf.at[pl.ds(0, step), :], acc.at[pl.ds(base + done, step), :])
        done += step


def _tile_row_split(N):
    ra = ((N // _NS + 7) // 8) * 8
    rlast = N - (_NS - 1) * ra
    assert rlast > 0 and rlast % 8 == 0
    return ra, rlast


def _make_seg0(N, E, Dh):
    nvreg = Dh // 16
    EP = E // _NS
    CPB = 5
    BE = CPB * _CH
    NB = EP // BE
    assert EP % BE == 0
    G = 4 if NB % 4 == 0 else (2 if NB % 2 == 0 else 1)
    NG = NB // G
    RA, RLAST = _tile_row_split(N)
    CROWS = EP // _CH

    mesh = plsc.VectorSubcoreMesh(core_axis_name="c", subcore_axis_name="s")

    @functools.partial(
        pl.kernel,
        mesh=mesh,
        out_type=(
            jax.ShapeDtypeStruct((N, Dh), jnp.float32),
            jax.ShapeDtypeStruct((N, Dh), jnp.float32),
        ),
        scratch_types=[
            pltpu.VMEM((G * BE,), jnp.int32),
            pltpu.VMEM((G * BE,), jnp.int32),
            pltpu.VMEM((G * BE,), jnp.float32),
            pltpu.VMEM((BE, Dh), jnp.float32),
            pltpu.MemorySpace.VMEM_SHARED((N, Dh), jnp.float32),
            pltpu.SemaphoreType.DMA,
            pltpu.SemaphoreType.DMA,
            pltpu.SemaphoreType.DMA,
        ],
        compiler_params=pltpu.CompilerParams(use_tc_tiling_on_sc=False),
    )
    def k(lo_hbm, hi_hbm, cols_hbm, rows_hbm, vals_hbm, out_lo, out_hi,
          cols_v, rows_v, vals_v, gbuf, acc, gsem, ssem, isem):
        c = lax.axis_index("c")
        s = lax.axis_index("s")

        def run(tab, out):
            _zero_gbuf(gbuf, BE, nvreg)
            rbase = pl.multiple_of(s * RA, 8)

            @pl.when(s < _NS - 1)
            def _():
                _fill_zero(gbuf, acc, rbase, RA, BE)

            @pl.when(s == _NS - 1)
            def _():
                _fill_zero(gbuf, acc, rbase, RLAST, BE)

            plsc.subcore_barrier()

            def grp(g, _):
                e0 = s * EP + g * (G * BE)
                ic = pltpu.async_copy(cols_hbm.at[pl.ds(e0, G * BE)], cols_v, isem)
                ir = pltpu.async_copy(rows_hbm.at[pl.ds(e0, G * BE)], rows_v, isem)
                iv = pltpu.async_copy(vals_hbm.at[pl.ds(e0, G * BE)], vals_v, isem)
                ic.wait()
                ir.wait()
                iv.wait()

                def blk(b, _):
                    vb = b * BE
                    gds = [
                        pltpu.async_copy(
                            tab.at[cols_v.at[pl.ds(vb + j * _CH, _CH)]],
                            gbuf.at[pl.ds(j * _CH, _CH), :],
                            gsem,
                        )
                        for j in range(CPB)
                    ]
                    sds = []
                    for j in range(CPB):
                        gds[j].wait()
                        _scale_chunk(gbuf, vals_v, vb, j, nvreg)
                        sds.append(
                            pltpu.async_copy(
                                gbuf.at[pl.ds(j * _CH, _CH), :],
                                acc.at[rows_v.at[pl.ds(vb + j * _CH, _CH)]],
                                ssem,
                                add=True,
                            )
                        )
                    for d in sds:
                        d.wait()
                    return 0

                lax.fori_loop(0, G, blk, 0)
                return 0

            lax.fori_loop(0, NG, grp, 0)
            plsc.subcore_barrier()

            @pl.when(s < _NS - 1)
            def _():
                pltpu.sync_copy(acc.at[pl.ds(rbase, RA), :], out.at[pl.ds(rbase, RA), :])

            @pl.when(s == _NS - 1)
            def _():
                pltpu.sync_copy(acc.at[pl.ds(rbase, RLAST), :], out.at[pl.ds(rbase, RLAST), :])

        @pl.when(c == 0)
        def _():
            run(lo_hbm, out_lo)

        @pl.when(c == 1)
        def _():
            run(hi_hbm, out_hi)

    return k


def _make_seg1(N, E, Din):
    nvreg = Din // 16
    NW = _NC * _NS
    EP = E // NW
    CPB = 5
    BE = CPB * _CH
    NB = EP // BE
    assert EP % BE == 0
    G = 4 if NB % 4 == 0 else (2 if NB % 2 == 0 else 1)
    NG = NB // G
    RA, RLAST = _tile_row_split(N)
    CROWS = EP // _CH

    mesh = plsc.VectorSubcoreMesh(core_axis_name="c", subcore_axis_name="s")

    @functools.partial(
        pl.kernel,
        mesh=mesh,
        out_type=(
            jax.ShapeDtypeStruct((N, Din), jnp.float32),
            jax.ShapeDtypeStruct((N, Din), jnp.float32),
        ),
        scratch_types=[
            pltpu.VMEM((G * BE,), jnp.int32),
            pltpu.VMEM((G * BE,), jnp.int32),
            pltpu.VMEM((G * BE,), jnp.float32),
            pltpu.VMEM((BE, Din), jnp.float32),
            pltpu.MemorySpace.VMEM_SHARED((N, Din), jnp.float32),
            pltpu.SemaphoreType.DMA,
            pltpu.SemaphoreType.DMA,
            pltpu.SemaphoreType.DMA,
        ],
        compiler_params=pltpu.CompilerParams(use_tc_tiling_on_sc=False),
    )
    def k(tab_hbm, cols_hbm, rows_hbm, vals_hbm, out_a, out_b,
          cols_v, rows_v, vals_v, gbuf, acc, gsem, ssem, isem):
        c = lax.axis_index("c")
        s = lax.axis_index("s")
        wid = s * _NC + c

        _zero_gbuf(gbuf, BE, nvreg)
        rbase = pl.multiple_of(s * RA, 8)

        @pl.when(s < _NS - 1)
        def _():
            _fill_zero(gbuf, acc, rbase, RA, BE)

        @pl.when(s == _NS - 1)
        def _():
            _fill_zero(gbuf, acc, rbase, RLAST, BE)

        plsc.subcore_barrier()

        def grp(g, _):
            e0 = wid * EP + g * (G * BE)
            ic = pltpu.async_copy(cols_hbm.at[pl.ds(e0, G * BE)], cols_v, isem)
            ir = pltpu.async_copy(rows_hbm.at[pl.ds(e0, G * BE)], rows_v, isem)
            iv = pltpu.async_copy(vals_hbm.at[pl.ds(e0, G * BE)], vals_v, isem)
            ic.wait()
            ir.wait()
            iv.wait()

            def blk(b, _):
                vb = b * BE
                gds = [
                    pltpu.async_copy(
                        tab_hbm.at[cols_v.at[pl.ds(vb + j * _CH, _CH)]],
                        gbuf.at[pl.ds(j * _CH, _CH), :],
                        gsem,
                    )
                    for j in range(CPB)
                ]
                sds = []
                for j in range(CPB):
                    gds[j].wait()
                    _scale_chunk(gbuf, vals_v, vb, j, nvreg)
                    sds.append(
                        pltpu.async_copy(
                            gbuf.at[pl.ds(j * _CH, _CH), :],
                            acc.at[rows_v.at[pl.ds(vb + j * _CH, _CH)]],
                            ssem,
                            add=True,
                        )
                    )
                for d in sds:
                    d.wait()
                return 0

            lax.fori_loop(0, G, blk, 0)
            return 0

        lax.fori_loop(0, NG, grp, 0)
        plsc.subcore_barrier()

        def wb(out):
            @pl.when(s < _NS - 1)
            def _():
                pltpu.sync_copy(acc.at[pl.ds(rbase, RA), :], out.at[pl.ds(rbase, RA), :])

            @pl.when(s == _NS - 1)
            def _():
                pltpu.sync_copy(acc.at[pl.ds(rbase, RLAST), :], out.at[pl.ds(rbase, RLAST), :])

        @pl.when(c == 0)
        def _():
            wb(out_a)

        @pl.when(c == 1)
        def _():
            wb(out_b)

    return k


def _make_gather(N, B):
    NW = _NC * _NS
    per_w = B // NW
    nch = per_w // 128
    assert per_w % 128 == 0

    mesh = plsc.VectorSubcoreMesh(core_axis_name="c", subcore_axis_name="s")

    @functools.partial(
        pl.kernel,
        mesh=mesh,
        out_type=jax.ShapeDtypeStruct((B, 160), jnp.float32),
        scratch_types=[
            pltpu.VMEM((nch, 128), jnp.int32),
            pltpu.VMEM((per_w, 64), jnp.float32),
            pltpu.VMEM((per_w, 32), jnp.float32),
            pltpu.VMEM((per_w, 32), jnp.float32),
            pltpu.VMEM((per_w, 32), jnp.float32),
            pltpu.SemaphoreType.DMA,
        ],
        compiler_params=pltpu.CompilerParams(use_tc_tiling_on_sc=False),
    )
    def k(t64, te1, tna, tnb, idx_hbm, out, idx_v, b64, be1, bna, bnb, sem):
        c = lax.axis_index("c")
        s = lax.axis_index("s")
        wid = s * _NC + c
        pltpu.sync_copy(idx_hbm.at[wid], idx_v)
        ds_ = []
        for j in range(nch):
            ds_.append(pltpu.async_copy(t64.at[idx_v.at[j]], b64.at[pl.ds(j * 128, 128), :], sem))
            ds_.append(pltpu.async_copy(te1.at[idx_v.at[j]], be1.at[pl.ds(j * 128, 128), :], sem))
            ds_.append(pltpu.async_copy(tna.at[idx_v.at[j]], bna.at[pl.ds(j * 128, 128), :], sem))
            ds_.append(pltpu.async_copy(tnb.at[idx_v.at[j]], bnb.at[pl.ds(j * 128, 128), :], sem))
        for d in ds_:
            d.wait()
        base = pl.multiple_of(wid * per_w, 8)
        pltpu.sync_copy(b64, out.at[pl.ds(base, per_w), pl.ds(0, 64)])
        pltpu.sync_copy(be1, out.at[pl.ds(base, per_w), pl.ds(64, 32)])
        pltpu.sync_copy(bna, out.at[pl.ds(base, per_w), pl.ds(96, 32)])
        pltpu.sync_copy(bnb, out.at[pl.ds(base, per_w), pl.ds(128, 32)])

    return k


def _leaky(x):
    return jnp.where(x > 0, x, x * _SLOPE)


def _dense0_body(ego, nlo, nhi, wg, bg, wb, bb, ego1_out):
    ego = ego[...]
    neigh = jnp.concatenate([nlo[...], nhi[...]], axis=1)
    x1 = ego + neigh
    x2 = ego * neigh
    a = _leaky(jnp.dot(x1, wg[...], preferred_element_type=jnp.float32) + bg[...])
    w = _leaky(jnp.dot(x2, wb[...], preferred_element_type=jnp.float32) + bb[...])
    ego1_out[...] = a + w


def _norm_rows(x):
    nrm = jnp.sqrt(jnp.sum(x * x, axis=1, keepdims=True))
    return x / jnp.maximum(nrm, 1e-12)


def _dense1_loss_body(packed, wg, bg, wb, bb, out):
    x = packed[...]
    g64 = x[:, :64]
    ge1 = x[:, 64:96]
    n1 = x[:, 96:128] + x[:, 128:160]
    g32 = _norm_rows(ge1)
    x1 = ge1 + n1
    x2 = ge1 * n1
    a = _leaky(jnp.dot(x1, wg[...], preferred_element_type=jnp.float32) + bg[...])
    w = _leaky(jnp.dot(x2, wb[...], preferred_element_type=jnp.float32) + bb[...])
    g16 = _norm_rows(a + w)
    B = g64.shape[0] // 3
    u64, p64, n64 = g64[:B], g64[B:2 * B], g64[2 * B:]
    u32, p32, n32 = g32[:B], g32[B:2 * B], g32[2 * B:]
    u16, p16, n16 = g16[:B], g16[B:2 * B], g16[2 * B:]
    pos = (jnp.sum(u64 * p64, axis=1, keepdims=True)
           + jnp.sum(u32 * p32, axis=1, keepdims=True)
           + jnp.sum(u16 * p16, axis=1, keepdims=True))
    neg = (jnp.sum(u64 * n64, axis=1, keepdims=True)
           + jnp.sum(u32 * n32, axis=1, keepdims=True)
           + jnp.sum(u16 * n16, axis=1, keepdims=True))
    d = pos - neg
    sp = jnp.maximum(-d, 0.0) + jnp.log1p(jnp.exp(-jnp.abs(d)))
    cf = jnp.sum(sp) / B
    sq_u = (jnp.sum(u64 * u64, axis=1, keepdims=True)
            + jnp.sum(u32 * u32, axis=1, keepdims=True)
            + jnp.sum(u16 * u16, axis=1, keepdims=True))
    sq_p = (jnp.sum(p64 * p64, axis=1, keepdims=True)
            + jnp.sum(p32 * p32, axis=1, keepdims=True)
            + jnp.sum(p16 * p16, axis=1, keepdims=True))
    sq_n = (jnp.sum(n64 * n64, axis=1, keepdims=True)
            + jnp.sum(n32 * n32, axis=1, keepdims=True)
            + jnp.sum(n16 * n16, axis=1, keepdims=True))
    l2 = (jnp.sum(sq_u) + jnp.sum(sq_p) + jnp.sum(sq_n)) / (2.0 * B)
    out[...] = jnp.reshape(cf + _CF_LAMBDA * l2, (1, 1))


def _dense0_call(N):
    R = 2000
    grid = (N // R,)
    full = lambda i: (0, 0)
    blk = lambda i: (i, 0)

    return pl.pallas_call(
        _dense0_body,
        grid=grid,
        in_specs=[
            pl.BlockSpec((R, 64), blk),
            pl.BlockSpec((R, 32), blk),
            pl.BlockSpec((R, 32), blk),
            pl.BlockSpec((64, 32), full),
            pl.BlockSpec((1, 32), full),
            pl.BlockSpec((64, 32), full),
            pl.BlockSpec((1, 32), full),
        ],
        out_specs=pl.BlockSpec((R, 32), blk),
        out_shape=jax.ShapeDtypeStruct((N, 32), jnp.float32),
    )


def _dense1_loss_call():
    return pl.pallas_call(
        _dense1_loss_body,
        out_shape=jax.ShapeDtypeStruct((1, 1), jnp.float32),
    )


def kernel(all_emb, W_gc0, b_gc0, W_bi0, b_bi0, W_gc1, b_gc1, W_bi1, b_bi1,
           edge_vals, user_ids, item_pos_ids, item_neg_ids, edge_index):
    N, D = all_emb.shape
    E = edge_vals.shape[0]
    Dh = D // 2

    align = _NC * _NS * _CH * 8
    Ep = ((E + align - 1) // align) * align
    pad = Ep - E
    rows = edge_index[0].astype(jnp.int32)
    cols = edge_index[1].astype(jnp.int32)
    if pad:
        spread = (jnp.arange(pad, dtype=jnp.int32) * 97) % jnp.int32(N)
        rows = jnp.concatenate([rows, spread])
        cols = jnp.concatenate([cols, spread])
        vals = jnp.concatenate([edge_vals, jnp.zeros((pad,), jnp.float32)])
    else:
        vals = edge_vals
    ego_lo = all_emb[:, :Dh]
    ego_hi = all_emb[:, Dh:]

    n_lo, n_hi = _make_seg0(N, Ep, Dh)(ego_lo, ego_hi, cols, rows, vals)
    ego1 = _dense0_call(N)(
        all_emb, n_lo, n_hi,
        W_gc0, b_gc0.reshape(1, -1), W_bi0, b_bi0.reshape(1, -1))

    n1a, n1b = _make_seg1(N, Ep, 32)(ego1, cols, rows, vals)

    B = user_ids.shape[0]
    idx = jnp.concatenate([
        user_ids.astype(jnp.int32),
        item_pos_ids.astype(jnp.int32) + _N_USERS,
        item_neg_ids.astype(jnp.int32) + _N_USERS,
    ]).reshape(_NC * _NS, (3 * B) // (_NC * _NS * 128), 128)

    packed = _make_gather(N, 3 * B)(all_emb, ego1, n1a, n1b, idx)
    loss = _dense1_loss_call()(
        packed, W_gc1, b_gc1.reshape(1, -1), W_bi1, b_bi1.reshape(1, -1))
    return loss.reshape(())


# speedup vs baseline: 1.4283x; 1.0276x over previous
import functools

import jax
import jax.numpy as jnp
from jax import lax
from jax.experimental import pallas as pl
from jax.experimental.pallas import tpu as pltpu
from jax.experimental.pallas import tpu_sc as plsc

_N_USERS = 10000
_CF_LAMBDA = 1e-05
_SLOPE = 0.01

_NC = 2
_NS = 16
_CH = 128


def _scale_chunk(gbuf, vals_v, vbase, j, nvreg):
    def body(k, _):
        v16 = vals_v[pl.ds(vbase + j * _CH + k * 16, 16)]
        for i in range(16):
            r = j * _CH + k * 16 + i
            v = v16[i]
            for q in range(nvreg):
                g = gbuf[r, pl.ds(q * 16, 16)]
                gbuf[r, pl.ds(q * 16, 16)] = g * v
        return 0

    lax.fori_loop(0, _CH // 16, body, 0)


def _zero_gbuf(gbuf, be, nvreg):
    z = jnp.zeros((16,), jnp.float32)

    def zb(i, _):
        for q in range(nvreg):
            gbuf[i, pl.ds(q * 16, 16)] = z
        return 0

    lax.fori_loop(0, be, zb, 0)


def _fill_zero(gbuf, acc, base, total, be):
    done = 0
    while done < total:
        step = min(be, total - done)
        pltpu.sync_copy(gbuf.at[pl.ds(0, step), :], acc.at[pl.ds(base + done, step), :])
        done += step


def _tile_row_split(N):
    ra = ((N // _NS + 7) // 8) * 8
    rlast = N - (_NS - 1) * ra
    assert rlast > 0 and rlast % 8 == 0
    return ra, rlast


def _pipelined_gather(tab, idx_v, gbuf, sem, nch, slots, copy_out):
    issued = {}
    for j in range(min(slots, nch)):
        issued[j] = pltpu.async_copy(
            tab.at[idx_v.at[pl.ds(j * 128, 128)]],
            gbuf.at[pl.ds((j % slots) * 128, 128), :],
            sem,
        )
    for j in range(nch):
        issued[j].wait()
        copy_out(j, gbuf.at[pl.ds((j % slots) * 128, 128), :])
        nxt = j + slots
        if nxt < nch:
            issued[nxt] = pltpu.async_copy(
                tab.at[idx_v.at[pl.ds(nxt * 128, 128)]],
                gbuf.at[pl.ds((j % slots) * 128, 128), :],
                sem,
            )


def _make_seg0(N, E, Dh, B3):
    nvreg = Dh // 16
    EP = E // _NS
    CPB = 5
    BE = CPB * _CH
    NB = EP // BE
    assert EP % BE == 0
    G = 4 if NB % 4 == 0 else (2 if NB % 2 == 0 else 1)
    NG = NB // G
    RA, RLAST = _tile_row_split(N)
    PB = B3 // _NS
    NCH = PB // 128
    assert PB % 128 == 0 and PB % 8 == 0
    SLOTS = min(BE // 128, NCH)

    mesh = plsc.VectorSubcoreMesh(core_axis_name="c", subcore_axis_name="s")

    @functools.partial(
        pl.kernel,
        mesh=mesh,
        out_type=(
            jax.ShapeDtypeStruct((N, Dh), jnp.float32),
            jax.ShapeDtypeStruct((N, Dh), jnp.float32),
            jax.ShapeDtypeStruct((B3, Dh), jnp.float32),
            jax.ShapeDtypeStruct((B3, Dh), jnp.float32),
        ),
        scratch_types=[
            pltpu.VMEM((G * BE,), jnp.int32),
            pltpu.VMEM((G * BE,), jnp.int32),
            pltpu.VMEM((G * BE,), jnp.float32),
            pltpu.VMEM((BE, Dh), jnp.float32),
            pltpu.MemorySpace.VMEM_SHARED((N, Dh), jnp.float32),
            pltpu.SemaphoreType.DMA,
            pltpu.SemaphoreType.DMA,
            pltpu.SemaphoreType.DMA,
            pltpu.VMEM((PB,), jnp.int32),
        ],
        compiler_params=pltpu.CompilerParams(use_tc_tiling_on_sc=False),
    )
    def k(lo_hbm, hi_hbm, cols_hbm, rows_hbm, vals_hbm, sidx_hbm,
          out_lo, out_hi, pk_lo, pk_hi,
          cols_v, rows_v, vals_v, gbuf, acc, gsem, ssem, isem, idxs):
        c = lax.axis_index("c")
        s = lax.axis_index("s")

        def run(tab, out, pk):
            _zero_gbuf(gbuf, BE, nvreg)
            rbase = pl.multiple_of(s * RA, 8)

            @pl.when(s < _NS - 1)
            def _():
                _fill_zero(gbuf, acc, rbase, RA, BE)

            @pl.when(s == _NS - 1)
            def _():
                _fill_zero(gbuf, acc, rbase, RLAST, BE)

            plsc.subcore_barrier()

            def grp(g, _):
                e0 = s * EP + g * (G * BE)
                ic = pltpu.async_copy(cols_hbm.at[pl.ds(e0, G * BE)], cols_v, isem)
                ir = pltpu.async_copy(rows_hbm.at[pl.ds(e0, G * BE)], rows_v, isem)
                iv = pltpu.async_copy(vals_hbm.at[pl.ds(e0, G * BE)], vals_v, isem)
                ic.wait()
                ir.wait()
                iv.wait()

                def blk(b, _):
                    vb = b * BE
                    gds = [
                        pltpu.async_copy(
                            tab.at[cols_v.at[pl.ds(vb + j * _CH, _CH)]],
                            gbuf.at[pl.ds(j * _CH, _CH), :],
                            gsem,
                        )
                        for j in range(CPB)
                    ]
                    sds = []
                    for j in range(CPB):
                        gds[j].wait()
                        _scale_chunk(gbuf, vals_v, vb, j, nvreg)
                        sds.append(
                            pltpu.async_copy(
                                gbuf.at[pl.ds(j * _CH, _CH), :],
                                acc.at[rows_v.at[pl.ds(vb + j * _CH, _CH)]],
                                ssem,
                                add=True,
                            )
                        )
                    for d in sds:
                        d.wait()
                    return 0

                lax.fori_loop(0, G, blk, 0)
                return 0

            lax.fori_loop(0, NG, grp, 0)
            plsc.subcore_barrier()

            @pl.when(s < _NS - 1)
            def _():
                pltpu.sync_copy(acc.at[pl.ds(rbase, RA), :], out.at[pl.ds(rbase, RA), :])

            @pl.when(s == _NS - 1)
            def _():
                pltpu.sync_copy(acc.at[pl.ds(rbase, RLAST), :], out.at[pl.ds(rbase, RLAST), :])

            gb = pl.multiple_of(s * PB, 8)
            pltpu.sync_copy(sidx_hbm.at[pl.ds(gb, PB)], idxs)
            _pipelined_gather(
                tab, idxs, gbuf, gsem, NCH, SLOTS,
                lambda j, src: pltpu.sync_copy(src, pk.at[pl.ds(gb + j * 128, 128), :]),
            )

        @pl.when(c == 0)
        def _():
            run(lo_hbm, out_lo, pk_lo)

        @pl.when(c == 1)
        def _():
            run(hi_hbm, out_hi, pk_hi)

    return k


def _make_seg1(N, E, Din, B3):
    nvreg = Din // 16
    NW = _NC * _NS
    EP = E // NW
    CPB = 5
    BE = CPB * _CH
    NB = EP // BE
    assert EP % BE == 0
    G = 4 if NB % 4 == 0 else (2 if NB % 2 == 0 else 1)
    NG = NB // G
    RA, RLAST = _tile_row_split(N)
    PB = B3 // _NS
    NCH = PB // 128
    PW = B3 // NW
    NCW = PW // 128
    assert PB % 128 == 0 and PW % 128 == 0
    SLOTS = min(BE // 128, NCH)

    mesh = plsc.VectorSubcoreMesh(core_axis_name="c", subcore_axis_name="s")

    @functools.partial(
        pl.kernel,
        mesh=mesh,
        out_type=(
            jax.ShapeDtypeStruct((B3, Din), jnp.float32),
            jax.ShapeDtypeStruct((B3, Din), jnp.float32),
            jax.ShapeDtypeStruct((B3, Din), jnp.float32),
        ),
        scratch_types=[
            pltpu.VMEM((G * BE,), jnp.int32),
            pltpu.VMEM((G * BE,), jnp.int32),
            pltpu.VMEM((G * BE,), jnp.float32),
            pltpu.VMEM((BE, Din), jnp.float32),
            pltpu.MemorySpace.VMEM_SHARED((N, Din), jnp.float32),
            pltpu.SemaphoreType.DMA,
            pltpu.SemaphoreType.DMA,
            pltpu.SemaphoreType.DMA,
            pltpu.VMEM((PB,), jnp.int32),
        ],
        compiler_params=pltpu.CompilerParams(use_tc_tiling_on_sc=False),
    )
    def k(tab_hbm, cols_hbm, rows_hbm, vals_hbm, sidx_hbm,
          pk_e1, pk_na, pk_nb,
          cols_v, rows_v, vals_v, gbuf, acc, gsem, ssem, isem, idxs):
        c = lax.axis_index("c")
        s = lax.axis_index("s")
        wid = s * _NC + c

        _zero_gbuf(gbuf, BE, nvreg)
        rbase = pl.multiple_of(s * RA, 8)

        @pl.when(s < _NS - 1)
        def _():
            _fill_zero(gbuf, acc, rbase, RA, BE)

        @pl.when(s == _NS - 1)
        def _():
            _fill_zero(gbuf, acc, rbase, RLAST, BE)

        plsc.subcore_barrier()

        def grp(g, _):
            e0 = wid * EP + g * (G * BE)
            ic = pltpu.async_copy(cols_hbm.at[pl.ds(e0, G * BE)], cols_v, isem)
            ir = pltpu.async_copy(rows_hbm.at[pl.ds(e0, G * BE)], rows_v, isem)
            iv = pltpu.async_copy(vals_hbm.at[pl.ds(e0, G * BE)], vals_v, isem)
            ic.wait()
            ir.wait()
            iv.wait()

            def blk(b, _):
                vb = b * BE
                gds = [
                    pltpu.async_copy(
                        tab_hbm.at[cols_v.at[pl.ds(vb + j * _CH, _CH)]],
                        gbuf.at[pl.ds(j * _CH, _CH), :],
                        gsem,
                    )
                    for j in range(CPB)
                ]
                sds = []
                for j in range(CPB):
                    gds[j].wait()
                    _scale_chunk(gbuf, vals_v, vb, j, nvreg)
                    sds.append(
                        pltpu.async_copy(
                            gbuf.at[pl.ds(j * _CH, _CH), :],
                            acc.at[rows_v.at[pl.ds(vb + j * _CH, _CH)]],
                            ssem,
                            add=True,
                        )
                    )
                for d in sds:
                    d.wait()
                return 0

            lax.fori_loop(0, G, blk, 0)
            return 0

        lax.fori_loop(0, NG, grp, 0)
        plsc.subcore_barrier()

        wb = pl.multiple_of(wid * PW, 8)
        pltpu.sync_copy(sidx_hbm.at[pl.ds(wb, PW)], idxs.at[pl.ds(0, PW)])
        _pipelined_gather(
            tab_hbm, idxs, gbuf, gsem, NCW, min(SLOTS, NCW),
            lambda j, src: pltpu.sync_copy(src, pk_e1.at[pl.ds(wb + j * 128, 128), :]),
        )

        gb = pl.multiple_of(s * PB, 8)
        pltpu.sync_copy(sidx_hbm.at[pl.ds(gb, PB)], idxs)

        def acc_gather(pk):
            _pipelined_gather(
                acc, idxs, gbuf, gsem, NCH, SLOTS,
                lambda j, src: pltpu.sync_copy(src, pk.at[pl.ds(gb + j * 128, 128), :]),
            )

        @pl.when(c == 0)
        def _():
            acc_gather(pk_na)

        @pl.when(c == 1)
        def _():
            acc_gather(pk_nb)

    return k


def _leaky(x):
    return jnp.where(x > 0, x, x * _SLOPE)


def _dense0_body(ego, nlo, nhi, wg, bg, wb, bb, ego1_out):
    ego = ego[...]
    neigh = jnp.concatenate([nlo[...], nhi[...]], axis=1)
    x1 = ego + neigh
    x2 = ego * neigh
    a = _leaky(jnp.dot(x1, wg[...], preferred_element_type=jnp.float32) + bg[...])
    w = _leaky(jnp.dot(x2, wb[...], preferred_element_type=jnp.float32) + bb[...])
    ego1_out[...] = a + w


def _norm_rows(x):
    nrm = jnp.sqrt(jnp.sum(x * x, axis=1, keepdims=True))
    return x / jnp.maximum(nrm, 1e-12)


def _dense1_loss_body(pka, pkb, pke, pna, pnb, wg, bg, wb, bb, out):
    pka_ = pka[...]
    pkb_ = pkb[...]
    ge1 = pke[...]
    n1 = pna[...] + pnb[...]
    g32 = _norm_rows(ge1)
    x1 = ge1 + n1
    x2 = ge1 * n1
    a = _leaky(jnp.dot(x1, wg[...], preferred_element_type=jnp.float32) + bg[...])
    w = _leaky(jnp.dot(x2, wb[...], preferred_element_type=jnp.float32) + bb[...])
    g16 = _norm_rows(a + w)
    B = pka_.shape[0] // 3

    def thirds(x):
        return x[:B], x[B:2 * B], x[2 * B:]

    ua, pa, na = thirds(pka_)
    ub, pb, nb = thirds(pkb_)
    u32, p32, n32 = thirds(g32)
    u16, p16, n16 = thirds(g16)

    def rdot(x, y):
        return jnp.sum(x * y, axis=1, keepdims=True)

    pos = (rdot(ua, pa) + rdot(ub, pb) + rdot(u32, p32) + rdot(u16, p16))
    neg = (rdot(ua, na) + rdot(ub, nb) + rdot(u32, n32) + rdot(u16, n16))
    d = pos - neg
    sp = jnp.maximum(-d, 0.0) + jnp.log1p(jnp.exp(-jnp.abs(d)))
    cf = jnp.sum(sp) / B
    sq_u = rdot(ua, ua) + rdot(ub, ub) + rdot(u32, u32) + rdot(u16, u16)
    sq_p = rdot(pa, pa) + rdot(pb, pb) + rdot(p32, p32) + rdot(p16, p16)
    sq_n = rdot(na, na) + rdot(nb, nb) + rdot(n32, n32) + rdot(n16, n16)
    l2 = (jnp.sum(sq_u) + jnp.sum(sq_p) + jnp.sum(sq_n)) / (2.0 * B)
    out[...] = jnp.reshape(cf + _CF_LAMBDA * l2, (1, 1))


def _dense0_call(N):
    R = 2000
    grid = (N // R,)
    full = lambda i: (0, 0)
    blk = lambda i: (i, 0)

    return pl.pallas_call(
        _dense0_body,
        grid=grid,
        in_specs=[
            pl.BlockSpec((R, 64), blk),
            pl.BlockSpec((R, 32), blk),
            pl.BlockSpec((R, 32), blk),
            pl.BlockSpec((64, 32), full),
            pl.BlockSpec((1, 32), full),
            pl.BlockSpec((64, 32), full),
            pl.BlockSpec((1, 32), full),
        ],
        out_specs=pl.BlockSpec((R, 32), blk),
        out_shape=jax.ShapeDtypeStruct((N, 32), jnp.float32),
    )


def _dense1_loss_call():
    return pl.pallas_call(
        _dense1_loss_body,
        out_shape=jax.ShapeDtypeStruct((1, 1), jnp.float32),
    )


def kernel(all_emb, W_gc0, b_gc0, W_bi0, b_bi0, W_gc1, b_gc1, W_bi1, b_bi1,
           edge_vals, user_ids, item_pos_ids, item_neg_ids, edge_index):
    N, D = all_emb.shape
    E = edge_vals.shape[0]
    Dh = D // 2

    align = _NC * _NS * _CH * 8
    Ep = ((E + align - 1) // align) * align
    pad = Ep - E
    rows = edge_index[0].astype(jnp.int32)
    cols = edge_index[1].astype(jnp.int32)
    if pad:
        spread = (jnp.arange(pad, dtype=jnp.int32) * 97) % jnp.int32(N)
        rows = jnp.concatenate([rows, spread])
        cols = jnp.concatenate([cols, spread])
        vals = jnp.concatenate([edge_vals, jnp.zeros((pad,), jnp.float32)])
    else:
        vals = edge_vals
    ego_lo = all_emb[:, :Dh]
    ego_hi = all_emb[:, Dh:]

    B = user_ids.shape[0]
    B3 = 3 * B
    sidx = jnp.concatenate([
        user_ids.astype(jnp.int32),
        item_pos_ids.astype(jnp.int32) + _N_USERS,
        item_neg_ids.astype(jnp.int32) + _N_USERS,
    ])

    n_lo, n_hi, pk64a, pk64b = _make_seg0(N, Ep, Dh, B3)(
        ego_lo, ego_hi, cols, rows, vals, sidx)
    ego1 = _dense0_call(N)(
        all_emb, n_lo, n_hi,
        W_gc0, b_gc0.reshape(1, -1), W_bi0, b_bi0.reshape(1, -1))

    pk_e1, pk_na, pk_nb = _make_seg1(N, Ep, 32, B3)(ego1, cols, rows, vals, sidx)

    loss = _dense1_loss_call()(
        pk64a, pk64b, pk_e1, pk_na, pk_nb,
        W_gc1, b_gc1.reshape(1, -1), W_bi1, b_bi1.reshape(1, -1))
    return loss.reshape(())
